# Initial kernel scaffold; baseline (speedup 1.0000x reference)
import jax
import jax.numpy as jnp
from jax.experimental import pallas as pl


def kernel(x, edge_index, y, W_g1, b_g1, W_g2, b_g2, W_i1, b_i1, W_i2, b_i2, W_e1, b_e1, W_e2, b_e2, W_d1, b_d1, W_d2, b_d2, g_ln, b_ln):
    raise NotImplementedError("write your pallas kernel here")


# speedup vs baseline: 13.0260x; 13.0260x over previous
import functools

import jax
import jax.numpy as jnp
from jax import lax
from jax.experimental import pallas as pl
from jax.experimental.pallas import tpu as pltpu
from jax.experimental.pallas import tpu_sc as plsc

_NC = 2
_NS = 16
_B = 128


def _sc_mesh():
    return plsc.VectorSubcoreMesh(
        core_axis_name="c", subcore_axis_name="s",
        num_cores=_NC, num_subcores=_NS)


def _pad_nodes(n_nodes):
    q = 8 * _NS
    return -(-n_nodes // q) * q


def _degree_partials(dst2d, n_nodes):
    R = dst2d.shape[0]
    RPW = -(-R // (_NC * _NS))
    npad = _pad_nodes(n_nodes)
    NPS = npad // _NS

    @functools.partial(
        pl.kernel,
        out_type=jax.ShapeDtypeStruct((_NC, npad, 16), jnp.float32),
        mesh=_sc_mesh(),
        compiler_params=pltpu.CompilerParams(use_tc_tiling_on_sc=False),
        scratch_types=[
            pltpu.VMEM((_B,), jnp.int32),
            pltpu.VMEM((_B, 16), jnp.float32),
            pltpu.VMEM_SHARED((npad, 16), jnp.float32),
        ],
    )
    def deg_kernel(dst_hbm, zeros_hbm, ones_hbm, out_hbm, idx_v, ones_v, dacc):
        c = lax.axis_index("c")
        s = lax.axis_index("s")
        wid = s * _NC + c
        pltpu.sync_copy(zeros_hbm, dacc.at[pl.ds(s * NPS, NPS)])
        pltpu.sync_copy(ones_hbm, ones_v)
        plsc.subcore_barrier()

        def body(j, carry):
            r = wid * RPW + j

            @pl.when(r < R)
            def _():
                pltpu.sync_copy(dst_hbm.at[r], idx_v)
                pltpu.sync_copy(ones_v, dacc.at[idx_v], add=True)

            return carry

        lax.fori_loop(0, RPW, body, 0)
        plsc.subcore_barrier()
        pltpu.sync_copy(dacc.at[pl.ds(s * NPS, NPS)],
                        out_hbm.at[c, pl.ds(s * NPS, NPS)])

    zeros = jnp.zeros((NPS, 16), jnp.float32)
    ones = jnp.ones((_B, 16), jnp.float32)
    return deg_kernel(dst2d, zeros, ones)


def _segsum_partials(table, src2d, dst2d):
    n_nodes, d = table.shape
    R = src2d.shape[0]
    RPW = -(-R // (_NC * _NS))
    npad = _pad_nodes(n_nodes)
    NPS = npad // _NS

    @functools.partial(
        pl.kernel,
        out_type=jax.ShapeDtypeStruct((_NC, npad, d), jnp.float32),
        mesh=_sc_mesh(),
        scratch_types=[
            pltpu.VMEM((_B,), jnp.int32),
            pltpu.VMEM((_B,), jnp.int32),
            pltpu.VMEM((_B, d), jnp.float32),
            pltpu.VMEM_SHARED((npad, d), jnp.float32),
            pltpu.SemaphoreType.DMA,
        ],
    )
    def seg_kernel(tab_hbm, src_hbm, dst_hbm, zeros_hbm, out_hbm,
                   sidx, didx, rows, sacc, sem):
        c = lax.axis_index("c")
        s = lax.axis_index("s")
        wid = s * _NC + c
        pltpu.sync_copy(zeros_hbm, sacc.at[pl.ds(s * NPS, NPS)])
        plsc.subcore_barrier()

        def body(j, carry):
            r = wid * RPW + j

            @pl.when(r < R)
            def _():
                pltpu.sync_copy(src_hbm.at[r], sidx)
                pltpu.sync_copy(dst_hbm.at[r], didx)
                pltpu.async_copy(tab_hbm.at[sidx], rows, sem).wait()
                pltpu.sync_copy(rows, sacc.at[didx], add=True)

            return carry

        lax.fori_loop(0, RPW, body, 0)
        plsc.subcore_barrier()
        pltpu.sync_copy(sacc.at[pl.ds(s * NPS, NPS)],
                        out_hbm.at[c, pl.ds(s * NPS, NPS)])

    zeros = jnp.zeros((NPS, d), jnp.float32)
    return seg_kernel(table, src2d, dst2d, zeros)


def _tc_prescale(deg0, deg1, x):
    n_nodes, d = x.shape

    def body(d0_ref, d1_ref, x_ref, xp_ref, dis_ref):
        deg = d0_ref[:, 0:1] + d1_ref[:, 0:1] + 1.0
        dis = lax.rsqrt(deg)
        xp_ref[...] = x_ref[...] * dis
        dis_ref[...] = dis

    return pl.pallas_call(
        body,
        out_shape=(jax.ShapeDtypeStruct((n_nodes, d), jnp.float32),
                   jax.ShapeDtypeStruct((n_nodes, 1), jnp.float32)),
    )(deg0, deg1, x)


def _tc_mid(s10, s11, xp, dis, W_g1, b_g1, W_i1, b_i1):
    n_nodes, d = xp.shape

    def body(s10_ref, s11_ref, xp_ref, dis_ref, wg_ref, bg_ref, wi_ref,
             bi_ref, out_ref):
        dis = dis_ref[...]
        agg = dis * (s10_ref[...] + s11_ref[...] + xp_ref[...])
        h = jnp.maximum(
            jnp.dot(agg, wg_ref[...], preferred_element_type=jnp.float32)
            + bg_ref[...], 0.0)
        hi = jnp.maximum(
            jnp.dot(agg, wi_ref[...], preferred_element_type=jnp.float32)
            + bi_ref[...], 0.0)
        out_ref[...] = jnp.concatenate([h, hi], axis=1) * dis

    return pl.pallas_call(
        body,
        out_shape=jax.ShapeDtypeStruct((n_nodes, d), jnp.float32),
    )(s10, s11, xp, dis, W_g1, b_g1.reshape(1, -1), W_i1, b_i1.reshape(1, -1))


def _tc_final(s20, s21, hc, dis, y2d, W_g2p, b_g2, W_i2p, b_i2,
              W_e1, b_e1, W_e2, b_e2, W_d1, b_d1, W_d2, b_d2, g_ln, b_ln):
    n_nodes = hc.shape[0]
    ne, _, c_cls = W_d2.shape

    def body(s20_ref, s21_ref, hc_ref, dis_ref, y_ref, wg_ref, bg_ref,
             wi_ref, bi_ref, we1_ref, be1_ref, we2_ref, be2_ref, wd1_ref,
             bd1_ref, wd2_ref, bd2_ref, gln_ref, bln_ref, out_ref):
        dis = dis_ref[...]
        aggc = dis * (s20_ref[...] + s21_ref[...] + hc_ref[...])
        out = (jnp.dot(aggc, wg_ref[...], preferred_element_type=jnp.float32)
               + bg_ref[...])
        ir = jnp.tanh(
            jnp.dot(aggc, wi_ref[...], preferred_element_type=jnp.float32)
            + bi_ref[...])
        e1 = jnp.maximum(
            jnp.dot(ir, we1_ref[...], preferred_element_type=jnp.float32)
            + be1_ref[...], 0.0)
        e_new = jnp.maximum(
            jnp.dot(e1, we2_ref[...], preferred_element_type=jnp.float32)
            + be2_ref[...], 0.0)
        me = jnp.mean(e_new, axis=0, keepdims=True)

        onehot = (lax.broadcasted_iota(jnp.int32, (n_nodes, c_cls), 1)
                  == y_ref[...]).astype(jnp.float32)

        total = jnp.float32(0.0)
        for i in range(ne):
            d1 = jnp.maximum(
                jnp.dot(out, wd1_ref[i],
                        preferred_element_type=jnp.float32)
                + bd1_ref[i:i + 1], 0.0)
            d2 = (jnp.dot(d1, wd2_ref[i],
                          preferred_element_type=jnp.float32)
                  + bd2_ref[i:i + 1])
            mu = jnp.mean(d2, axis=-1, keepdims=True)
            var = jnp.mean((d2 - mu) ** 2, axis=-1, keepdims=True)
            dif = ((d2 - mu) / jnp.sqrt(var + 1e-5) * gln_ref[i:i + 1]
                   + bln_ref[i:i + 1])
            m = jnp.max(dif, axis=-1, keepdims=True)
            lse = jnp.log(jnp.sum(jnp.exp(dif - m), axis=-1,
                                  keepdims=True)) + m
            logp = dif - lse
            loss = -jnp.sum(logp * onehot) / n_nodes
            total = total + loss * me[0, i]
        out_ref[...] = jnp.reshape(total / ne, (1, 1))

    return pl.pallas_call(
        body,
        out_shape=jax.ShapeDtypeStruct((1, 1), jnp.float32),
    )(s20, s21, hc, dis, y2d, W_g2p, b_g2.reshape(1, -1), W_i2p,
      b_i2.reshape(1, -1), W_e1, b_e1.reshape(1, -1), W_e2,
      b_e2.reshape(1, -1), W_d1, b_d1, W_d2, b_d2, g_ln, b_ln)


def kernel(x, edge_index, y, W_g1, b_g1, W_g2, b_g2, W_i1, b_i1, W_i2, b_i2,
           W_e1, b_e1, W_e2, b_e2, W_d1, b_d1, W_d2, b_d2, g_ln, b_ln):
    n_nodes, d = x.shape
    e = edge_index.shape[1]
    h = W_g2.shape[0]

    src2d = edge_index[0].reshape(e // _B, _B)
    dst2d = edge_index[1].reshape(e // _B, _B)

    degp = _degree_partials(dst2d, n_nodes)
    xp, dis = _tc_prescale(degp[0, :n_nodes], degp[1, :n_nodes], x)

    s1 = _segsum_partials(xp, src2d, dst2d)
    hc = _tc_mid(s1[0, :n_nodes], s1[1, :n_nodes], xp, dis,
                 W_g1, b_g1, W_i1, b_i1)

    s2 = _segsum_partials(hc, src2d, dst2d)

    zero_h = jnp.zeros((h, h), jnp.float32)
    W_g2p = jnp.concatenate([W_g2, zero_h], axis=0)
    W_i2p = jnp.concatenate([zero_h, W_i2], axis=0)

    loss = _tc_final(s2[0, :n_nodes], s2[1, :n_nodes], hc, dis,
                     y.reshape(n_nodes, 1),
                     W_g2p, b_g2, W_i2p, b_i2, W_e1, b_e1, W_e2, b_e2,
                     W_d1, b_d1, W_d2, b_d2, g_ln, b_ln)
    return loss[0, 0]
